# Initial kernel scaffold; baseline (speedup 1.0000x reference)
import functools

import jax
import jax.numpy as jnp
from jax import lax
from jax.experimental import pallas as pl
from jax.experimental.pallas import tpu as pltpu
from jax.experimental.pallas import tpu_sc as plsc

R0 = 5.0
RANK = 16


def _prep_nodes_body(emb_ref, z0_ref, z1_ref, wsrc_ref, wdst_ref, wg_ref,
                     wz0_ref, wz1_ref, a_ref, dtab_ref):
    wg = wg_ref[...]
    wsg = jnp.dot(wsrc_ref[...], wg, preferred_element_type=jnp.float32)
    wdg = jnp.dot(wdst_ref[...], wg, preferred_element_type=jnp.float32)
    emb = emb_ref[...]
    a_ref[...] = jnp.dot(emb, wsg, preferred_element_type=jnp.float32)
    dtab_ref[:, 0:16] = jnp.dot(emb, wdg, preferred_element_type=jnp.float32)
    dtab_ref[:, 16:32] = jnp.dot(z0_ref[...], wz0_ref[...],
                                 preferred_element_type=jnp.float32)
    wz1 = wz1_ref[...]
    for k in range(3):
        dtab_ref[:, 32 + 16 * k:48 + 16 * k] = jnp.dot(
            z1_ref[:, k, :], wz1, preferred_element_type=jnp.float32)


def _prep_nodes(emb, z_0, z_1, w_src, w_dst, w_g, w_z0, w_z1):
    n = emb.shape[0]
    nb = 2000
    grid = (n // nb,)
    full = lambda shape: pl.BlockSpec(shape, lambda i: tuple(0 for _ in shape))
    return pl.pallas_call(
        _prep_nodes_body,
        grid=grid,
        in_specs=[
            pl.BlockSpec((nb, 128), lambda i: (i, 0)),
            pl.BlockSpec((nb, 128), lambda i: (i, 0)),
            pl.BlockSpec((nb, 3, 128), lambda i: (i, 0, 0)),
            full((128, 128)),
            full((128, 128)),
            full((128, RANK)),
            full((128, RANK)),
            full((128, RANK)),
        ],
        out_specs=[
            pl.BlockSpec((nb, RANK), lambda i: (i, 0)),
            pl.BlockSpec((nb, 80), lambda i: (i, 0)),
        ],
        out_shape=[
            jax.ShapeDtypeStruct((n, RANK), jnp.float32),
            jax.ShapeDtypeStruct((n, 80), jnp.float32),
        ],
    )(emb, z_0, z_1, w_src, w_dst, w_g, w_z0, w_z1)


def _prep_edges_body(r_ref, wenc_ref, wg_ref, benc_ref, bsrc_ref, bdst_ref,
                     encg_ref, rsq_ref):
    r = r_ref[...]
    d = jnp.sqrt(jnp.sum(r * r, axis=1, keepdims=True) + 1e-12)
    eb = r.shape[0]
    n_idx = lax.broadcasted_iota(jnp.float32, (eb, 8), 1) + 1.0
    radial = jnp.sin((jnp.pi / R0) * d * n_idx)
    wg = wg_ref[...]
    weg = jnp.dot(wenc_ref[...], wg, preferred_element_type=jnp.float32)
    bsum = benc_ref[...] + bsrc_ref[...] + bdst_ref[...]
    bg = jnp.dot(bsum, wg, preferred_element_type=jnp.float32)
    encg_ref[...] = jnp.dot(radial, weg,
                            preferred_element_type=jnp.float32) + bg
    u = r * (7.0 / R0)
    rsq_ref[...] = u * lax.rsqrt(1.0 + jnp.sum(u * u, axis=1, keepdims=True))


def _prep_edges(r_ij, w_enc, w_g, b_enc, b_src, b_dst):
    e = r_ij.shape[0]
    eb = 2000
    grid = (e // eb,)
    full = lambda shape: pl.BlockSpec(shape, lambda i: tuple(0 for _ in shape))
    return pl.pallas_call(
        _prep_edges_body,
        grid=grid,
        in_specs=[
            pl.BlockSpec((eb, 3), lambda i: (i, 0)),
            full((8, 128)),
            full((128, RANK)),
            full((1, 128)),
            full((1, 128)),
            full((1, 128)),
        ],
        out_specs=[
            pl.BlockSpec((eb, RANK), lambda i: (i, 0)),
            pl.BlockSpec((eb, 3), lambda i: (i, 0)),
        ],
        out_shape=[
            jax.ShapeDtypeStruct((e, RANK), jnp.float32),
            jax.ShapeDtypeStruct((e, 3), jnp.float32),
        ],
    )(r_ij, w_enc, w_g,
      b_enc.reshape(1, 128), b_src.reshape(1, 128), b_dst.reshape(1, 128))


_C = 128


def _sc_edges(a_tab, d_tab, encg, rsq, src, dst, zeros):
    n = a_tab.shape[0]
    e = src.shape[0]
    info = plsc.get_sparse_core_info()
    nc, ns = info.num_cores, info.num_subcores
    nw = nc * ns
    nch = e // _C
    max_j = (nch + nw - 1) // nw
    rows_per = n // ns

    @functools.partial(
        pl.kernel,
        out_type=jax.ShapeDtypeStruct((nc, n, 64), jnp.float32),
        mesh=plsc.VectorSubcoreMesh(core_axis_name="c", subcore_axis_name="s"),
        scratch_types=[
            pltpu.VMEM((_C,), jnp.int32),
            pltpu.VMEM((_C,), jnp.int32),
            pltpu.VMEM((_C, 16), jnp.float32),
            pltpu.VMEM((_C, 3), jnp.float32),
            pltpu.VMEM((_C, 16), jnp.float32),
            pltpu.VMEM((_C, 80), jnp.float32),
            pltpu.VMEM((_C, 64), jnp.float32),
            pltpu.VMEM_SHARED((n, 64), jnp.float32),
            pltpu.SemaphoreType.DMA,
            pltpu.SemaphoreType.DMA,
        ],
    )
    def k(a_hbm, d_hbm, encg_hbm, rsq_hbm, src_hbm, dst_hbm, z_hbm, part_hbm,
          src_v, dst_v, encg_v, rsq_v, a_v, d_v, out_v, acc, sem_a, sem_d):
        c = lax.axis_index("c")
        s = lax.axis_index("s")
        wid = s * nc + c
        pltpu.sync_copy(z_hbm, acc.at[pl.ds(s * rows_per, rows_per)])
        plsc.subcore_barrier()

        col0 = jnp.zeros((16,), jnp.int32)
        col1 = jnp.full((16,), 1, jnp.int32)
        col2 = jnp.full((16,), 2, jnp.int32)

        def chunk(j, carry):
            ci = wid + nw * j

            @pl.when(ci < nch)
            def _():
                base = ci * _C
                pltpu.sync_copy(src_hbm.at[pl.ds(base, _C)], src_v)
                pltpu.sync_copy(dst_hbm.at[pl.ds(base, _C)], dst_v)
                pltpu.sync_copy(encg_hbm.at[pl.ds(base, _C)], encg_v)
                pltpu.sync_copy(rsq_hbm.at[pl.ds(base, _C)], rsq_v)
                ga = pltpu.async_copy(a_hbm.at[src_v], a_v, sem_a)
                gd = pltpu.async_copy(d_hbm.at[dst_v], d_v, sem_d)
                ga.wait()
                gd.wait()

                def inner(ei, carry2):
                    evec = jnp.full((16,), ei, jnp.int32)
                    g = encg_v[ei, :] + a_v[ei, :] + d_v[ei, 0:16]
                    z0 = d_v[ei, 16:32]
                    v0 = d_v[ei, 32:48]
                    v1 = d_v[ei, 48:64]
                    v2 = d_v[ei, 64:80]
                    r0 = plsc.load_gather(rsq_v, [evec, col0])
                    r1 = plsc.load_gather(rsq_v, [evec, col1])
                    r2 = plsc.load_gather(rsq_v, [evec, col2])
                    us = z0 + r0 * v0 + r1 * v1 + r2 * v2
                    out_v[ei, 0:16] = us * g
                    out_v[ei, 16:32] = (v0 + r0 * z0) * g
                    out_v[ei, 32:48] = (v1 + r1 * z0) * g
                    out_v[ei, 48:64] = (v2 + r2 * z0) * g
                    return carry2

                lax.fori_loop(0, _C, inner, 0)
                pltpu.sync_copy(out_v, acc.at[src_v], add=True)

            return carry

        lax.fori_loop(0, max_j, chunk, 0)
        plsc.subcore_barrier()
        pltpu.sync_copy(acc.at[pl.ds(s * rows_per, rows_per)],
                        part_hbm.at[c, pl.ds(s * rows_per, rows_per)])

    return k(a_tab, d_tab, encg, rsq, src, dst, zeros)


def _post_body(part_ref, wo0_ref, wo1_ref, out0_ref, out1_ref):
    acc = part_ref[0] + part_ref[1]
    out0_ref[...] = jnp.dot(acc[:, 0:16], wo0_ref[...],
                            preferred_element_type=jnp.float32)
    wo1 = wo1_ref[...]
    for k in range(3):
        out1_ref[:, k, :] = jnp.dot(acc[:, 16 * (k + 1):16 * (k + 2)], wo1,
                                    preferred_element_type=jnp.float32)


def _post(part, w_o0, w_o1):
    n = part.shape[1]
    nb = 2000
    grid = (n // nb,)
    full = lambda shape: pl.BlockSpec(shape, lambda i: tuple(0 for _ in shape))
    return pl.pallas_call(
        _post_body,
        grid=grid,
        in_specs=[
            pl.BlockSpec((2, nb, 64), lambda i: (0, i, 0)),
            full((RANK, 128)),
            full((RANK, 128)),
        ],
        out_specs=[
            pl.BlockSpec((nb, 128), lambda i: (i, 0)),
            pl.BlockSpec((nb, 3, 128), lambda i: (i, 0, 0)),
        ],
        out_shape=[
            jax.ShapeDtypeStruct((n, 128), jnp.float32),
            jax.ShapeDtypeStruct((n, 3, 128), jnp.float32),
        ],
    )(part, w_o0, w_o1)


def kernel(graph, r_ij, z_0, z_1, emb, W_enc, b_enc, W_src, b_src, W_dst,
           b_dst, W_z0, W_z1, W_g, W_o0, W_o1):
    src = graph[0]
    dst = graph[1]
    n = z_0.shape[0]
    a_tab, d_tab = _prep_nodes(emb, z_0, z_1, W_src, W_dst, W_g, W_z0, W_z1)
    encg, rsq = _prep_edges(r_ij, W_enc, W_g, b_enc, b_src, b_dst)
    zeros = jnp.zeros((n // 16, 64), jnp.float32)
    part = _sc_edges(a_tab, d_tab, encg, rsq, src, dst, zeros)
    return _post(part, W_o0, W_o1)


# speedup vs baseline: 26.2934x; 26.2934x over previous
import functools

import jax
import jax.numpy as jnp
from jax import lax
from jax.experimental import pallas as pl
from jax.experimental.pallas import tpu as pltpu
from jax.experimental.pallas import tpu_sc as plsc

R0 = 5.0
RANK = 16


def _prep_nodes_body(emb_ref, z0_ref, z1_ref, wsrc_ref, wdst_ref, wg_ref,
                     wz0_ref, wz1_ref, a_ref, dtab_ref):
    wg = wg_ref[...]
    wsg = jnp.dot(wsrc_ref[...], wg, preferred_element_type=jnp.float32)
    wdg = jnp.dot(wdst_ref[...], wg, preferred_element_type=jnp.float32)
    emb = emb_ref[...]
    a_ref[...] = jnp.dot(emb, wsg, preferred_element_type=jnp.float32)
    dtab_ref[:, 0:16] = jnp.dot(emb, wdg, preferred_element_type=jnp.float32)
    dtab_ref[:, 16:32] = jnp.dot(z0_ref[...], wz0_ref[...],
                                 preferred_element_type=jnp.float32)
    wz1 = wz1_ref[...]
    for k in range(3):
        dtab_ref[:, 32 + 16 * k:48 + 16 * k] = jnp.dot(
            z1_ref[:, k, :], wz1, preferred_element_type=jnp.float32)


def _prep_nodes(emb, z_0, z_1, w_src, w_dst, w_g, w_z0, w_z1):
    n = emb.shape[0]
    nb = 2000
    grid = (n // nb,)
    full = lambda shape: pl.BlockSpec(shape, lambda i: tuple(0 for _ in shape))
    return pl.pallas_call(
        _prep_nodes_body,
        grid=grid,
        in_specs=[
            pl.BlockSpec((nb, 128), lambda i: (i, 0)),
            pl.BlockSpec((nb, 128), lambda i: (i, 0)),
            pl.BlockSpec((nb, 3, 128), lambda i: (i, 0, 0)),
            full((128, 128)),
            full((128, 128)),
            full((128, RANK)),
            full((128, RANK)),
            full((128, RANK)),
        ],
        out_specs=[
            pl.BlockSpec((nb, RANK), lambda i: (i, 0)),
            pl.BlockSpec((nb, 80), lambda i: (i, 0)),
        ],
        out_shape=[
            jax.ShapeDtypeStruct((n, RANK), jnp.float32),
            jax.ShapeDtypeStruct((n, 80), jnp.float32),
        ],
    )(emb, z_0, z_1, w_src, w_dst, w_g, w_z0, w_z1)


def _prep_edges_body(r_ref, wenc_ref, wg_ref, benc_ref, bsrc_ref, bdst_ref,
                     feat_ref):
    r = r_ref[...]
    d = jnp.sqrt(jnp.sum(r * r, axis=1, keepdims=True) + 1e-12)
    eb = r.shape[0]
    n_idx = (lax.broadcasted_iota(jnp.int32, (eb, 8), 1) + 1).astype(jnp.float32)
    radial = jnp.sin((jnp.pi / R0) * d * n_idx)
    wg = wg_ref[...]
    weg = jnp.dot(wenc_ref[...], wg, preferred_element_type=jnp.float32)
    bsum = benc_ref[...] + bsrc_ref[...] + bdst_ref[...]
    bg = jnp.dot(bsum, wg, preferred_element_type=jnp.float32)
    feat_ref[:, 0:16] = jnp.dot(radial, weg,
                                preferred_element_type=jnp.float32) + bg
    u = r * (7.0 / R0)
    rsq = u * lax.rsqrt(1.0 + jnp.sum(u * u, axis=1, keepdims=True))
    feat_ref[:, 16:32] = jnp.concatenate(
        [rsq, jnp.zeros((eb, 13), jnp.float32)], axis=1)


def _prep_edges(r_ij, w_enc, w_g, b_enc, b_src, b_dst):
    e = r_ij.shape[0]
    eb = 2000
    grid = (e // eb,)
    full = lambda shape: pl.BlockSpec(shape, lambda i: tuple(0 for _ in shape))
    return pl.pallas_call(
        _prep_edges_body,
        grid=grid,
        in_specs=[
            pl.BlockSpec((eb, 3), lambda i: (i, 0)),
            full((8, 128)),
            full((128, RANK)),
            full((1, 128)),
            full((1, 128)),
            full((1, 128)),
        ],
        out_specs=pl.BlockSpec((eb, 32), lambda i: (i, 0)),
        out_shape=jax.ShapeDtypeStruct((e, 32), jnp.float32),
    )(r_ij, w_enc, w_g,
      b_enc.reshape(1, 128), b_src.reshape(1, 128), b_dst.reshape(1, 128))


_C = 128


def _sc_edges(a_tab, d_tab, feat, src, dst, zeros):
    e = src.shape[0]
    info = plsc.get_sparse_core_info()
    nc, ns = info.num_cores, info.num_subcores
    nw = nc * ns
    nch = e // _C
    max_j = (nch + nw - 1) // nw
    rows_per = zeros.shape[0]
    n = rows_per * ns

    @functools.partial(
        pl.kernel,
        out_type=jax.ShapeDtypeStruct((nc, n, 64), jnp.float32),
        mesh=plsc.VectorSubcoreMesh(core_axis_name="c", subcore_axis_name="s"),
        compiler_params=pltpu.CompilerParams(use_tc_tiling_on_sc=False),
        scratch_types=[
            pltpu.VMEM((_C,), jnp.int32),
            pltpu.VMEM((_C,), jnp.int32),
            pltpu.VMEM((_C, 32), jnp.float32),
            pltpu.VMEM((_C, 16), jnp.float32),
            pltpu.VMEM((_C, 80), jnp.float32),
            pltpu.VMEM((_C, 64), jnp.float32),
            pltpu.VMEM_SHARED((n, 64), jnp.float32),
            pltpu.SemaphoreType.DMA,
            pltpu.SemaphoreType.DMA,
        ],
    )
    def k(a_hbm, d_hbm, feat_hbm, src_hbm, dst_hbm, z_hbm, part_hbm,
          src_v, dst_v, feat_v, a_v, d_v, out_v, acc, sem_a, sem_d):
        c = lax.axis_index("c")
        s = lax.axis_index("s")
        wid = s * nc + c
        pltpu.sync_copy(z_hbm, acc.at[pl.ds(s * rows_per, rows_per)])
        plsc.subcore_barrier()

        def chunk(j, carry):
            ci = wid + nw * j

            @pl.when(ci < nch)
            def _():
                base = ci * _C
                pltpu.sync_copy(src_hbm.at[pl.ds(base, _C)], src_v)
                pltpu.sync_copy(dst_hbm.at[pl.ds(base, _C)], dst_v)
                pltpu.sync_copy(feat_hbm.at[pl.ds(base, _C)], feat_v)
                ga = pltpu.async_copy(a_hbm.at[src_v], a_v, sem_a)
                gd = pltpu.async_copy(d_hbm.at[dst_v], d_v, sem_d)
                ga.wait()
                gd.wait()

                def inner(ei, carry2):
                    g = feat_v[ei, 0:16] + a_v[ei, :] + d_v[ei, 0:16]
                    z0 = d_v[ei, 16:32]
                    v0 = d_v[ei, 32:48]
                    v1 = d_v[ei, 48:64]
                    v2 = d_v[ei, 64:80]
                    rv = feat_v[ei, 16:32]
                    r0 = jnp.full((16,), rv[0], jnp.float32)
                    r1 = jnp.full((16,), rv[1], jnp.float32)
                    r2 = jnp.full((16,), rv[2], jnp.float32)
                    us = z0 + r0 * v0 + r1 * v1 + r2 * v2
                    out_v[ei, 0:16] = us * g
                    out_v[ei, 16:32] = (v0 + r0 * z0) * g
                    out_v[ei, 32:48] = (v1 + r1 * z0) * g
                    out_v[ei, 48:64] = (v2 + r2 * z0) * g
                    return carry2

                lax.fori_loop(0, _C, inner, 0)
                pltpu.sync_copy(out_v, acc.at[src_v], add=True)

            return carry

        lax.fori_loop(0, max_j, chunk, 0)
        plsc.subcore_barrier()
        pltpu.sync_copy(acc.at[pl.ds(s * rows_per, rows_per)],
                        part_hbm.at[c, pl.ds(s * rows_per, rows_per)])

    return k(a_tab, d_tab, feat, src, dst, zeros)


def _post_body(part_ref, wo0_ref, wo1_ref, out0_ref, out1_ref):
    acc = part_ref[0] + part_ref[1]
    out0_ref[...] = jnp.dot(acc[:, 0:16], wo0_ref[...],
                            preferred_element_type=jnp.float32)
    wo1 = wo1_ref[...]
    for k in range(3):
        out1_ref[:, k, :] = jnp.dot(acc[:, 16 * (k + 1):16 * (k + 2)], wo1,
                                    preferred_element_type=jnp.float32)


def _post(part, w_o0, w_o1, n):
    npad = part.shape[1]
    nb = 2048
    grid = (npad // nb,)
    full = lambda shape: pl.BlockSpec(shape, lambda i: tuple(0 for _ in shape))
    return pl.pallas_call(
        _post_body,
        grid=grid,
        in_specs=[
            pl.BlockSpec((2, nb, 64), lambda i: (0, i, 0)),
            full((RANK, 128)),
            full((RANK, 128)),
        ],
        out_specs=[
            pl.BlockSpec((nb, 128), lambda i: (i, 0)),
            pl.BlockSpec((nb, 3, 128), lambda i: (i, 0, 0)),
        ],
        out_shape=[
            jax.ShapeDtypeStruct((n, 128), jnp.float32),
            jax.ShapeDtypeStruct((n, 3, 128), jnp.float32),
        ],
    )(part, w_o0, w_o1)


def kernel(graph, r_ij, z_0, z_1, emb, W_enc, b_enc, W_src, b_src, W_dst,
           b_dst, W_z0, W_z1, W_g, W_o0, W_o1):
    src = graph[0]
    dst = graph[1]
    n = z_0.shape[0]
    a_tab, d_tab = _prep_nodes(emb, z_0, z_1, W_src, W_dst, W_g, W_z0, W_z1)
    feat = _prep_edges(r_ij, W_enc, W_g, b_enc, b_src, b_dst)
    npad = -(-n // 2048) * 2048
    zeros = jnp.zeros((npad // 16, 64), jnp.float32)
    part = _sc_edges(a_tab, d_tab, feat, src, dst, zeros)
    return _post(part, W_o0, W_o1, n)


# speedup vs baseline: 44.3176x; 1.6855x over previous
import functools

import jax
import jax.numpy as jnp
from jax import lax
from jax.experimental import pallas as pl
from jax.experimental.pallas import tpu as pltpu
from jax.experimental.pallas import tpu_sc as plsc

R0 = 5.0
RANK = 16


def _prep_nodes_body(emb_ref, z0_ref, z1_ref, wsrc_ref, wdst_ref, wg_ref,
                     wz0_ref, wz1_ref, a_ref, dtab_ref):
    wg = wg_ref[...]
    wsg = jnp.dot(wsrc_ref[...], wg, preferred_element_type=jnp.float32)
    wdg = jnp.dot(wdst_ref[...], wg, preferred_element_type=jnp.float32)
    emb = emb_ref[...]
    a_ref[...] = jnp.dot(emb, wsg, preferred_element_type=jnp.float32)
    dtab_ref[:, 0:16] = jnp.dot(emb, wdg, preferred_element_type=jnp.float32)
    dtab_ref[:, 16:32] = jnp.dot(z0_ref[...], wz0_ref[...],
                                 preferred_element_type=jnp.float32)
    wz1 = wz1_ref[...]
    for k in range(3):
        dtab_ref[:, 32 + 16 * k:48 + 16 * k] = jnp.dot(
            z1_ref[:, k, :], wz1, preferred_element_type=jnp.float32)


def _prep_nodes(emb, z_0, z_1, w_src, w_dst, w_g, w_z0, w_z1):
    n = emb.shape[0]
    nb = 2000
    grid = (n // nb,)
    full = lambda shape: pl.BlockSpec(shape, lambda i: tuple(0 for _ in shape))
    return pl.pallas_call(
        _prep_nodes_body,
        grid=grid,
        in_specs=[
            pl.BlockSpec((nb, 128), lambda i: (i, 0)),
            pl.BlockSpec((nb, 128), lambda i: (i, 0)),
            pl.BlockSpec((nb, 3, 128), lambda i: (i, 0, 0)),
            full((128, 128)),
            full((128, 128)),
            full((128, RANK)),
            full((128, RANK)),
            full((128, RANK)),
        ],
        out_specs=[
            pl.BlockSpec((nb, RANK), lambda i: (i, 0)),
            pl.BlockSpec((nb, 80), lambda i: (i, 0)),
        ],
        out_shape=[
            jax.ShapeDtypeStruct((n, RANK), jnp.float32),
            jax.ShapeDtypeStruct((n, 80), jnp.float32),
        ],
    )(emb, z_0, z_1, w_src, w_dst, w_g, w_z0, w_z1)


def _prep_edges_body(rt_ref, wenc_ref, wg_ref, benc_ref, bsrc_ref, bdst_ref,
                     feat_ref):
    rt = rt_ref[...]
    eb = rt.shape[1]
    d = jnp.sqrt(jnp.sum(rt * rt, axis=0, keepdims=True) + 1e-12)
    n_idx = (lax.broadcasted_iota(jnp.int32, (8, 1), 0) + 1).astype(jnp.float32)
    radial_t = jnp.sin((jnp.pi / R0) * n_idx * d)
    wg = wg_ref[...]
    weg = jnp.dot(wenc_ref[...], wg, preferred_element_type=jnp.float32)
    bsum = benc_ref[...] + bsrc_ref[...] + bdst_ref[...]
    bg = jnp.dot(bsum, wg, preferred_element_type=jnp.float32)
    feat_ref[:, 0:16] = lax.dot_general(
        radial_t, weg, (((0,), (0,)), ((), ())),
        preferred_element_type=jnp.float32) + bg
    ut = rt * (7.0 / R0)
    rsq_t = ut * lax.rsqrt(1.0 + jnp.sum(ut * ut, axis=0, keepdims=True))
    eye3 = (lax.broadcasted_iota(jnp.int32, (3, 3), 0) ==
            lax.broadcasted_iota(jnp.int32, (3, 3), 1)).astype(jnp.float32)
    feat_ref[:, 16:19] = lax.dot_general(
        rsq_t, eye3, (((0,), (0,)), ((), ())),
        preferred_element_type=jnp.float32)
    feat_ref[:, 19:32] = jnp.zeros((eb, 13), jnp.float32)


def _prep_edges(r_t, w_enc, w_g, b_enc, b_src, b_dst):
    e = r_t.shape[1]
    eb = 3200
    grid = (e // eb,)
    full = lambda shape: pl.BlockSpec(shape, lambda i: tuple(0 for _ in shape))
    return pl.pallas_call(
        _prep_edges_body,
        grid=grid,
        in_specs=[
            pl.BlockSpec((3, eb), lambda i: (0, i)),
            full((8, 128)),
            full((128, RANK)),
            full((1, 128)),
            full((1, 128)),
            full((1, 128)),
        ],
        out_specs=pl.BlockSpec((eb, 32), lambda i: (i, 0)),
        out_shape=jax.ShapeDtypeStruct((e, 32), jnp.float32),
    )(r_t, w_enc, w_g,
      b_enc.reshape(1, 128), b_src.reshape(1, 128), b_dst.reshape(1, 128))


_C = 128


def _sc_edges(a_tab, d_tab, feat, src, dst, zeros):
    e = src.shape[0]
    info = plsc.get_sparse_core_info()
    nc, ns = info.num_cores, info.num_subcores
    nw = nc * ns
    nch = e // _C
    max_j = (nch + nw - 1) // nw
    rows_per = zeros.shape[0]
    n = rows_per * ns

    @functools.partial(
        pl.kernel,
        out_type=jax.ShapeDtypeStruct((nc, n, 64), jnp.float32),
        mesh=plsc.VectorSubcoreMesh(core_axis_name="c", subcore_axis_name="s"),
        compiler_params=pltpu.CompilerParams(use_tc_tiling_on_sc=False),
        scratch_types=[
            pltpu.VMEM((_C,), jnp.int32),
            pltpu.VMEM((_C,), jnp.int32),
            pltpu.VMEM((_C, 32), jnp.float32),
            pltpu.VMEM((_C, 16), jnp.float32),
            pltpu.VMEM((_C, 80), jnp.float32),
            pltpu.VMEM((_C, 64), jnp.float32),
            pltpu.VMEM_SHARED((n, 64), jnp.float32),
            pltpu.SemaphoreType.DMA,
            pltpu.SemaphoreType.DMA,
        ],
    )
    def k(a_hbm, d_hbm, feat_hbm, src_hbm, dst_hbm, z_hbm, part_hbm,
          src_v, dst_v, feat_v, a_v, d_v, out_v, acc, sem_a, sem_d):
        c = lax.axis_index("c")
        s = lax.axis_index("s")
        wid = s * nc + c
        pltpu.sync_copy(z_hbm, acc.at[pl.ds(s * rows_per, rows_per)])
        plsc.subcore_barrier()

        def chunk(j, carry):
            ci = wid + nw * j

            @pl.when(ci < nch)
            def _():
                base = ci * _C
                pltpu.sync_copy(src_hbm.at[pl.ds(base, _C)], src_v)
                pltpu.sync_copy(dst_hbm.at[pl.ds(base, _C)], dst_v)
                pltpu.sync_copy(feat_hbm.at[pl.ds(base, _C)], feat_v)
                ga = pltpu.async_copy(a_hbm.at[src_v], a_v, sem_a)
                gd = pltpu.async_copy(d_hbm.at[dst_v], d_v, sem_d)
                ga.wait()
                gd.wait()

                def inner(ei, carry2):
                    g = feat_v[ei, 0:16] + a_v[ei, :] + d_v[ei, 0:16]
                    z0 = d_v[ei, 16:32]
                    v0 = d_v[ei, 32:48]
                    v1 = d_v[ei, 48:64]
                    v2 = d_v[ei, 64:80]
                    rv = feat_v[ei, 16:32]
                    r0 = jnp.full((16,), rv[0], jnp.float32)
                    r1 = jnp.full((16,), rv[1], jnp.float32)
                    r2 = jnp.full((16,), rv[2], jnp.float32)
                    us = z0 + r0 * v0 + r1 * v1 + r2 * v2
                    out_v[ei, 0:16] = us * g
                    out_v[ei, 16:32] = (v0 + r0 * z0) * g
                    out_v[ei, 32:48] = (v1 + r1 * z0) * g
                    out_v[ei, 48:64] = (v2 + r2 * z0) * g
                    return carry2

                lax.fori_loop(0, _C, inner, 0)
                pltpu.sync_copy(out_v, acc.at[src_v], add=True)

            return carry

        lax.fori_loop(0, max_j, chunk, 0)
        plsc.subcore_barrier()
        pltpu.sync_copy(acc.at[pl.ds(s * rows_per, rows_per)],
                        part_hbm.at[c, pl.ds(s * rows_per, rows_per)])

    return k(a_tab, d_tab, feat, src, dst, zeros)


def _post_body(part_ref, wo0_ref, wo1_ref, out0_ref, out1_ref):
    acc = part_ref[0] + part_ref[1]
    out0_ref[...] = jnp.dot(acc[:, 0:16], wo0_ref[...],
                            preferred_element_type=jnp.float32)
    wo1 = wo1_ref[...]
    for k in range(3):
        out1_ref[:, k, :] = jnp.dot(acc[:, 16 * (k + 1):16 * (k + 2)], wo1,
                                    preferred_element_type=jnp.float32)


def _post(part, w_o0, w_o1, n):
    npad = part.shape[1]
    nb = 2048
    grid = (npad // nb,)
    full = lambda shape: pl.BlockSpec(shape, lambda i: tuple(0 for _ in shape))
    return pl.pallas_call(
        _post_body,
        grid=grid,
        in_specs=[
            pl.BlockSpec((2, nb, 64), lambda i: (0, i, 0)),
            full((RANK, 128)),
            full((RANK, 128)),
        ],
        out_specs=[
            pl.BlockSpec((nb, 128), lambda i: (i, 0)),
            pl.BlockSpec((nb, 3, 128), lambda i: (i, 0, 0)),
        ],
        out_shape=[
            jax.ShapeDtypeStruct((n, 128), jnp.float32),
            jax.ShapeDtypeStruct((n, 3, 128), jnp.float32),
        ],
    )(part, w_o0, w_o1)


def kernel(graph, r_ij, z_0, z_1, emb, W_enc, b_enc, W_src, b_src, W_dst,
           b_dst, W_z0, W_z1, W_g, W_o0, W_o1):
    src = graph[0]
    dst = graph[1]
    n = z_0.shape[0]
    a_tab, d_tab = _prep_nodes(emb, z_0, z_1, W_src, W_dst, W_g, W_z0, W_z1)
    feat = _prep_edges(r_ij.T, W_enc, W_g, b_enc, b_src, b_dst)
    npad = -(-n // 2048) * 2048
    zeros = jnp.zeros((npad // 16, 64), jnp.float32)
    part = _sc_edges(a_tab, d_tab, feat, src, dst, zeros)
    return _post(part, W_o0, W_o1, n)


# speedup vs baseline: 58.4631x; 1.3192x over previous
import functools

import jax
import jax.numpy as jnp
from jax import lax
from jax.experimental import pallas as pl
from jax.experimental.pallas import tpu as pltpu
from jax.experimental.pallas import tpu_sc as plsc

R0 = 5.0
RANK = 16


def _prep_nodes_body(emb_ref, z0_ref, z1_ref, wsrc_ref, wdst_ref, wg_ref,
                     wz0_ref, wz1_ref, a_ref, dtab_ref):
    wg = wg_ref[...]
    wsg = jnp.dot(wsrc_ref[...], wg, preferred_element_type=jnp.float32)
    wdg = jnp.dot(wdst_ref[...], wg, preferred_element_type=jnp.float32)
    emb = emb_ref[...]
    a_ref[...] = jnp.dot(emb, wsg, preferred_element_type=jnp.float32)
    dtab_ref[:, 0:16] = jnp.dot(emb, wdg, preferred_element_type=jnp.float32)
    dtab_ref[:, 16:32] = jnp.dot(z0_ref[...], wz0_ref[...],
                                 preferred_element_type=jnp.float32)
    wz1 = wz1_ref[...]
    for k in range(3):
        dtab_ref[:, 32 + 16 * k:48 + 16 * k] = jnp.dot(
            z1_ref[:, k, :], wz1, preferred_element_type=jnp.float32)


def _prep_nodes(emb, z_0, z_1, w_src, w_dst, w_g, w_z0, w_z1):
    n = emb.shape[0]
    nb = 2000
    grid = (n // nb,)
    full = lambda shape: pl.BlockSpec(shape, lambda i: tuple(0 for _ in shape))
    return pl.pallas_call(
        _prep_nodes_body,
        grid=grid,
        in_specs=[
            pl.BlockSpec((nb, 128), lambda i: (i, 0)),
            pl.BlockSpec((nb, 128), lambda i: (i, 0)),
            pl.BlockSpec((nb, 3, 128), lambda i: (i, 0, 0)),
            full((128, 128)),
            full((128, 128)),
            full((128, RANK)),
            full((128, RANK)),
            full((128, RANK)),
        ],
        out_specs=[
            pl.BlockSpec((nb, RANK), lambda i: (i, 0)),
            pl.BlockSpec((nb, 80), lambda i: (i, 0)),
        ],
        out_shape=[
            jax.ShapeDtypeStruct((n, RANK), jnp.float32),
            jax.ShapeDtypeStruct((n, 80), jnp.float32),
        ],
    )(emb, z_0, z_1, w_src, w_dst, w_g, w_z0, w_z1)


def _prep_edges_body(rt_ref, wenc_ref, wg_ref, benc_ref, bsrc_ref, bdst_ref,
                     feat_ref):
    rt = rt_ref[...]
    eb = rt.shape[1]
    d = jnp.sqrt(jnp.sum(rt * rt, axis=0, keepdims=True) + 1e-12)
    n_idx = (lax.broadcasted_iota(jnp.int32, (8, 1), 0) + 1).astype(jnp.float32)
    radial_t = jnp.sin((jnp.pi / R0) * n_idx * d)
    wg = wg_ref[...]
    weg = jnp.dot(wenc_ref[...], wg, preferred_element_type=jnp.float32)
    bsum = benc_ref[...] + bsrc_ref[...] + bdst_ref[...]
    bg = jnp.dot(bsum, wg, preferred_element_type=jnp.float32)
    feat_ref[:, 0:16] = lax.dot_general(
        radial_t, weg, (((0,), (0,)), ((), ())),
        preferred_element_type=jnp.float32) + bg
    ut = rt * (7.0 / R0)
    rsq_t = ut * lax.rsqrt(1.0 + jnp.sum(ut * ut, axis=0, keepdims=True))
    eye3 = (lax.broadcasted_iota(jnp.int32, (3, 3), 0) ==
            lax.broadcasted_iota(jnp.int32, (3, 3), 1)).astype(jnp.float32)
    feat_ref[:, 16:19] = lax.dot_general(
        rsq_t, eye3, (((0,), (0,)), ((), ())),
        preferred_element_type=jnp.float32)
    feat_ref[:, 19:32] = jnp.zeros((eb, 13), jnp.float32)
    feat_ref[:, 32:128] = jnp.zeros((eb, 96), jnp.float32)


def _prep_edges(r_t, w_enc, w_g, b_enc, b_src, b_dst):
    e = r_t.shape[1]
    eb = 3200
    grid = (e // eb,)
    full = lambda shape: pl.BlockSpec(shape, lambda i: tuple(0 for _ in shape))
    return pl.pallas_call(
        _prep_edges_body,
        grid=grid,
        in_specs=[
            pl.BlockSpec((3, eb), lambda i: (0, i)),
            full((8, 128)),
            full((128, RANK)),
            full((1, 128)),
            full((1, 128)),
            full((1, 128)),
        ],
        out_specs=pl.BlockSpec((eb, 128), lambda i: (i, 0)),
        out_shape=jax.ShapeDtypeStruct((e, 128), jnp.float32),
    )(r_t, w_enc, w_g,
      b_enc.reshape(1, 128), b_src.reshape(1, 128), b_dst.reshape(1, 128))


_C = 128


def _sc_edges(a_tab, d_tab, feat, src, dst, zeros):
    e = src.shape[0]
    info = plsc.get_sparse_core_info()
    nc, ns = info.num_cores, info.num_subcores
    nw = nc * ns
    per_w = e // nw
    nfull = per_w // _C
    tail = per_w - nfull * _C
    assert nfull % 2 == 1 and nfull >= 3 and tail % 8 == 0
    rows_per = zeros.shape[0]
    n = rows_per * ns

    @functools.partial(
        pl.kernel,
        out_type=jax.ShapeDtypeStruct((nc, n, 64), jnp.float32),
        mesh=plsc.VectorSubcoreMesh(core_axis_name="c", subcore_axis_name="s"),
        compiler_params=pltpu.CompilerParams(use_tc_tiling_on_sc=False),
        scratch_types=[
            pltpu.VMEM((_C,), jnp.int32),
            pltpu.VMEM((_C,), jnp.int32),
            pltpu.VMEM((_C,), jnp.int32),
            pltpu.VMEM((_C,), jnp.int32),
            pltpu.VMEM((_C, 128), jnp.float32),
            pltpu.VMEM((_C, 128), jnp.float32),
            pltpu.VMEM((_C, 16), jnp.float32),
            pltpu.VMEM((_C, 16), jnp.float32),
            pltpu.VMEM((_C, 80), jnp.float32),
            pltpu.VMEM((_C, 80), jnp.float32),
            pltpu.VMEM((_C, 64), jnp.float32),
            pltpu.VMEM((max(tail, 8),), jnp.int32),
            pltpu.VMEM_SHARED((n, 64), jnp.float32),
            pltpu.SemaphoreType.DMA,
            pltpu.SemaphoreType.DMA,
            pltpu.SemaphoreType.DMA,
            pltpu.SemaphoreType.DMA,
        ],
    )
    def k(a_hbm, d_hbm, feat_hbm, src_hbm, dst_hbm, z_hbm, part_hbm,
          src0, src1, dst0, dst1, feat0, feat1, a0, a1, d0, d1, out_v,
          src_t, acc, sl0, sl1, sg0, sg1):
        c = lax.axis_index("c")
        s = lax.axis_index("s")
        wid = s * nc + c
        base_w = wid * per_w
        bufs = ((src0, dst0, feat0, a0, d0, sl0, sg0),
                (src1, dst1, feat1, a1, d1, sl1, sg1))

        pltpu.sync_copy(z_hbm, acc.at[pl.ds(s * rows_per, rows_per)])
        plsc.subcore_barrier()

        def lin_copies(j, b):
            sv, dv, fv, _, _, sl, _ = bufs[b]
            base = base_w + j * _C
            return (
                pltpu.make_async_copy(src_hbm.at[pl.ds(base, _C)], sv, sl),
                pltpu.make_async_copy(dst_hbm.at[pl.ds(base, _C)], dv, sl),
                pltpu.make_async_copy(feat_hbm.at[pl.ds(base, _C)], fv, sl),
            )

        def gat_copies(b):
            sv, dv, _, av, dvv, _, sg = bufs[b]
            return (
                pltpu.make_async_copy(a_hbm.at[sv], av, sg),
                pltpu.make_async_copy(d_hbm.at[dv], dvv, sg),
            )

        def edge_math(fv, av, dvv, out_ref, ei):
            g = fv[ei, 0:16] + av[ei, :] + dvv[ei, 0:16]
            z0 = dvv[ei, 16:32]
            v0 = dvv[ei, 32:48]
            v1 = dvv[ei, 48:64]
            v2 = dvv[ei, 64:80]
            rv = fv[ei, 16:32]
            r0 = jnp.full((16,), rv[0], jnp.float32)
            r1 = jnp.full((16,), rv[1], jnp.float32)
            r2 = jnp.full((16,), rv[2], jnp.float32)
            us = z0 + r0 * v0 + r1 * v1 + r2 * v2
            out_ref[ei, 0:16] = us * g
            out_ref[ei, 16:32] = (v0 + r0 * z0) * g
            out_ref[ei, 32:48] = (v1 + r1 * z0) * g
            out_ref[ei, 48:64] = (v2 + r2 * z0) * g

        def compute_scatter(b, nedges, idx_ref=None):
            sv, _, fv, av, dvv, _, _ = bufs[b]

            def quad(i2, carry):
                for u in range(4):
                    edge_math(fv, av, dvv, out_v, i2 * 4 + u)
                return carry

            lax.fori_loop(0, nedges // 4, quad, 0)
            if nedges == _C:
                pltpu.sync_copy(out_v, acc.at[sv], add=True)
            else:
                pltpu.sync_copy(out_v.at[pl.ds(0, nedges)],
                                acc.at[idx_ref], add=True)

        for cp in lin_copies(0, 0):
            cp.start()
        for cp in lin_copies(1, 1):
            cp.start()
        for cp in lin_copies(0, 0):
            cp.wait()
        for cp in gat_copies(0):
            cp.start()

        def outer(i, carry):
            for b in (0, 1):
                j = 2 * i + b
                for cp in lin_copies(j + 1, 1 - b):
                    cp.wait()
                for cp in gat_copies(1 - b):
                    cp.start()
                for cp in gat_copies(b):
                    cp.wait()
                compute_scatter(b, _C)

                @pl.when(j <= nfull - 3)
                def _():
                    for cp in lin_copies(j + 2, b):
                        cp.start()
            return carry

        lax.fori_loop(0, (nfull - 1) // 2, outer, 0)
        lastb = (nfull - 1) % 2
        for cp in gat_copies(lastb):
            cp.wait()
        compute_scatter(lastb, _C)
        if tail:
            tb = base_w + nfull * _C
            pltpu.sync_copy(src_hbm.at[pl.ds(tb, tail)], src_t)
            pltpu.sync_copy(dst_hbm.at[pl.ds(tb, tail)],
                            dst0.at[pl.ds(0, tail)])
            pltpu.sync_copy(feat_hbm.at[pl.ds(tb, tail)],
                            feat0.at[pl.ds(0, tail)])
            pltpu.async_copy(a_hbm.at[src_t], a0.at[pl.ds(0, tail)],
                             sg0).wait()
            pltpu.async_copy(d_hbm.at[dst0.at[pl.ds(0, tail)]],
                             d0.at[pl.ds(0, tail)], sg0).wait()
            compute_scatter(0, tail, idx_ref=src_t)

        plsc.subcore_barrier()
        pltpu.sync_copy(acc.at[pl.ds(s * rows_per, rows_per)],
                        part_hbm.at[c, pl.ds(s * rows_per, rows_per)])

    return k(a_tab, d_tab, feat, src, dst, zeros)


def _post_body(part_ref, wo0_ref, wo1_ref, out0_ref, out1_ref):
    acc = part_ref[0] + part_ref[1]
    out0_ref[...] = jnp.dot(acc[:, 0:16], wo0_ref[...],
                            preferred_element_type=jnp.float32)
    wo1 = wo1_ref[...]
    for k in range(3):
        out1_ref[:, k, :] = jnp.dot(acc[:, 16 * (k + 1):16 * (k + 2)], wo1,
                                    preferred_element_type=jnp.float32)


def _post(part, w_o0, w_o1, n):
    npad = part.shape[1]
    nb = 2048
    grid = (npad // nb,)
    full = lambda shape: pl.BlockSpec(shape, lambda i: tuple(0 for _ in shape))
    return pl.pallas_call(
        _post_body,
        grid=grid,
        in_specs=[
            pl.BlockSpec((2, nb, 64), lambda i: (0, i, 0)),
            full((RANK, 128)),
            full((RANK, 128)),
        ],
        out_specs=[
            pl.BlockSpec((nb, 128), lambda i: (i, 0)),
            pl.BlockSpec((nb, 3, 128), lambda i: (i, 0, 0)),
        ],
        out_shape=[
            jax.ShapeDtypeStruct((n, 128), jnp.float32),
            jax.ShapeDtypeStruct((n, 3, 128), jnp.float32),
        ],
    )(part, w_o0, w_o1)


def kernel(graph, r_ij, z_0, z_1, emb, W_enc, b_enc, W_src, b_src, W_dst,
           b_dst, W_z0, W_z1, W_g, W_o0, W_o1):
    src = graph[0]
    dst = graph[1]
    n = z_0.shape[0]
    a_tab, d_tab = _prep_nodes(emb, z_0, z_1, W_src, W_dst, W_g, W_z0, W_z1)
    feat = _prep_edges(r_ij.T, W_enc, W_g, b_enc, b_src, b_dst)
    npad = -(-n // 2048) * 2048
    zeros = jnp.zeros((npad // 16, 64), jnp.float32)
    part = _sc_edges(a_tab, d_tab, feat, src, dst, zeros)
    return _post(part, W_o0, W_o1, n)


# speedup vs baseline: 66.8179x; 1.1429x over previous
import functools

import jax
import jax.numpy as jnp
from jax import lax
from jax.experimental import pallas as pl
from jax.experimental.pallas import tpu as pltpu
from jax.experimental.pallas import tpu_sc as plsc

R0 = 5.0
RANK = 16


def _prep_nodes_body(emb_ref, z0_ref, z1a_ref, z1b_ref, z1c_ref, wsrc_ref,
                     wdst_ref, wg_ref, wz0_ref, wz1_ref, a_ref, dtab_ref):
    wg = wg_ref[...]
    wsg = jnp.dot(wsrc_ref[...], wg, preferred_element_type=jnp.float32)
    wdg = jnp.dot(wdst_ref[...], wg, preferred_element_type=jnp.float32)
    emb = emb_ref[...]
    a_ref[...] = jnp.dot(emb, wsg, preferred_element_type=jnp.float32)
    dtab_ref[:, 0:16] = jnp.dot(emb, wdg, preferred_element_type=jnp.float32)
    dtab_ref[:, 16:32] = jnp.dot(z0_ref[...], wz0_ref[...],
                                 preferred_element_type=jnp.float32)
    wz1 = wz1_ref[...]
    for k, zr in enumerate((z1a_ref, z1b_ref, z1c_ref)):
        dtab_ref[:, 32 + 16 * k:48 + 16 * k] = jnp.dot(
            zr[...], wz1, preferred_element_type=jnp.float32)


def _prep_nodes(emb, z_0, z1a, z1b, z1c, w_src, w_dst, w_g, w_z0, w_z1):
    n = emb.shape[0]
    nb = 2000
    grid = (n // nb,)
    full = lambda shape: pl.BlockSpec(shape, lambda i: tuple(0 for _ in shape))
    row = pl.BlockSpec((nb, 128), lambda i: (i, 0))
    return pl.pallas_call(
        _prep_nodes_body,
        grid=grid,
        in_specs=[
            row, row, row, row, row,
            full((128, 128)),
            full((128, 128)),
            full((128, RANK)),
            full((128, RANK)),
            full((128, RANK)),
        ],
        out_specs=[
            pl.BlockSpec((nb, RANK), lambda i: (i, 0)),
            pl.BlockSpec((nb, 80), lambda i: (i, 0)),
        ],
        out_shape=[
            jax.ShapeDtypeStruct((n, RANK), jnp.float32),
            jax.ShapeDtypeStruct((n, 80), jnp.float32),
        ],
    )(emb, z_0, z1a, z1b, z1c, w_src, w_dst, w_g, w_z0, w_z1)


def _prep_edges_body(rt_ref, wenc_ref, wg_ref, benc_ref, bsrc_ref, bdst_ref,
                     feat_ref):
    rt = rt_ref[...]
    eb = rt.shape[1]
    d = jnp.sqrt(jnp.sum(rt * rt, axis=0, keepdims=True) + 1e-12)
    n_idx = (lax.broadcasted_iota(jnp.int32, (8, 1), 0) + 1).astype(jnp.float32)
    radial_t = jnp.sin((jnp.pi / R0) * n_idx * d)
    wg = wg_ref[...]
    weg = jnp.dot(wenc_ref[...], wg, preferred_element_type=jnp.float32)
    bsum = benc_ref[...] + bsrc_ref[...] + bdst_ref[...]
    bg = jnp.dot(bsum, wg, preferred_element_type=jnp.float32)
    feat_ref[:, 0:16] = lax.dot_general(
        radial_t, weg, (((0,), (0,)), ((), ())),
        preferred_element_type=jnp.float32) + bg
    ut = rt * (7.0 / R0)
    rsq_t = ut * lax.rsqrt(1.0 + jnp.sum(ut * ut, axis=0, keepdims=True))
    eye3 = (lax.broadcasted_iota(jnp.int32, (3, 3), 0) ==
            lax.broadcasted_iota(jnp.int32, (3, 3), 1)).astype(jnp.float32)
    feat_ref[:, 16:19] = lax.dot_general(
        rsq_t, eye3, (((0,), (0,)), ((), ())),
        preferred_element_type=jnp.float32)
    feat_ref[:, 19:32] = jnp.zeros((eb, 13), jnp.float32)
    feat_ref[:, 32:128] = jnp.zeros((eb, 96), jnp.float32)


def _prep_edges(r_t, w_enc, w_g, b_enc, b_src, b_dst):
    e = r_t.shape[1]
    eb = 3200
    grid = (e // eb,)
    full = lambda shape: pl.BlockSpec(shape, lambda i: tuple(0 for _ in shape))
    return pl.pallas_call(
        _prep_edges_body,
        grid=grid,
        in_specs=[
            pl.BlockSpec((3, eb), lambda i: (0, i)),
            full((8, 128)),
            full((128, RANK)),
            full((1, 128)),
            full((1, 128)),
            full((1, 128)),
        ],
        out_specs=pl.BlockSpec((eb, 128), lambda i: (i, 0)),
        out_shape=jax.ShapeDtypeStruct((e, 128), jnp.float32),
    )(r_t, w_enc, w_g,
      b_enc.reshape(1, 128), b_src.reshape(1, 128), b_dst.reshape(1, 128))


_C = 128


def _sc_edges(a_tab, d_tab, feat, src, dst, zeros):
    e = src.shape[0]
    info = plsc.get_sparse_core_info()
    nc, ns = info.num_cores, info.num_subcores
    nw = nc * ns
    per_w = e // nw
    nfull = per_w // _C
    tail = per_w - nfull * _C
    assert nfull % 2 == 1 and nfull >= 3 and tail % 8 == 0
    rows_per = zeros.shape[0]
    n = rows_per * ns

    @functools.partial(
        pl.kernel,
        out_type=jax.ShapeDtypeStruct((nc, n, 64), jnp.float32),
        mesh=plsc.VectorSubcoreMesh(core_axis_name="c", subcore_axis_name="s"),
        compiler_params=pltpu.CompilerParams(use_tc_tiling_on_sc=False),
        scratch_types=[
            pltpu.VMEM((_C,), jnp.int32),
            pltpu.VMEM((_C,), jnp.int32),
            pltpu.VMEM((_C,), jnp.int32),
            pltpu.VMEM((_C,), jnp.int32),
            pltpu.VMEM((_C, 128), jnp.float32),
            pltpu.VMEM((_C, 128), jnp.float32),
            pltpu.VMEM((_C, 16), jnp.float32),
            pltpu.VMEM((_C, 16), jnp.float32),
            pltpu.VMEM((_C, 80), jnp.float32),
            pltpu.VMEM((_C, 80), jnp.float32),
            pltpu.VMEM((_C, 64), jnp.float32),
            pltpu.VMEM((max(tail, 8),), jnp.int32),
            pltpu.VMEM_SHARED((n, 64), jnp.float32),
            pltpu.SemaphoreType.DMA,
            pltpu.SemaphoreType.DMA,
            pltpu.SemaphoreType.DMA,
            pltpu.SemaphoreType.DMA,
        ],
    )
    def k(a_hbm, d_hbm, feat_hbm, src_hbm, dst_hbm, z_hbm, part_hbm,
          src0, src1, dst0, dst1, feat0, feat1, a0, a1, d0, d1, out_v,
          src_t, acc, sl0, sl1, sg0, sg1):
        c = lax.axis_index("c")
        s = lax.axis_index("s")
        wid = s * nc + c
        base_w = wid * per_w
        bufs = ((src0, dst0, feat0, a0, d0, sl0, sg0),
                (src1, dst1, feat1, a1, d1, sl1, sg1))

        pltpu.sync_copy(z_hbm, acc.at[pl.ds(s * rows_per, rows_per)])
        plsc.subcore_barrier()

        def lin_copies(j, b):
            sv, dv, fv, _, _, sl, _ = bufs[b]
            base = base_w + j * _C
            return (
                pltpu.make_async_copy(src_hbm.at[pl.ds(base, _C)], sv, sl),
                pltpu.make_async_copy(dst_hbm.at[pl.ds(base, _C)], dv, sl),
                pltpu.make_async_copy(feat_hbm.at[pl.ds(base, _C)], fv, sl),
            )

        def gat_copies(b):
            sv, dv, _, av, dvv, _, sg = bufs[b]
            return (
                pltpu.make_async_copy(a_hbm.at[sv], av, sg),
                pltpu.make_async_copy(d_hbm.at[dv], dvv, sg),
            )

        def edge_math(fv, av, dvv, out_ref, ei):
            g = fv[ei, 0:16] + av[ei, :] + dvv[ei, 0:16]
            z0 = dvv[ei, 16:32]
            v0 = dvv[ei, 32:48]
            v1 = dvv[ei, 48:64]
            v2 = dvv[ei, 64:80]
            rv = fv[ei, 16:32]
            r0 = jnp.full((16,), rv[0], jnp.float32)
            r1 = jnp.full((16,), rv[1], jnp.float32)
            r2 = jnp.full((16,), rv[2], jnp.float32)
            us = z0 + r0 * v0 + r1 * v1 + r2 * v2
            out_ref[ei, 0:16] = us * g
            out_ref[ei, 16:32] = (v0 + r0 * z0) * g
            out_ref[ei, 32:48] = (v1 + r1 * z0) * g
            out_ref[ei, 48:64] = (v2 + r2 * z0) * g

        def compute_scatter(b, nedges, idx_ref=None):
            sv, _, fv, av, dvv, _, _ = bufs[b]

            def quad(i2, carry):
                for u in range(4):
                    edge_math(fv, av, dvv, out_v, i2 * 4 + u)
                return carry

            lax.fori_loop(0, nedges // 4, quad, 0)
            if nedges == _C:
                pltpu.sync_copy(out_v, acc.at[sv], add=True)
            else:
                pltpu.sync_copy(out_v.at[pl.ds(0, nedges)],
                                acc.at[idx_ref], add=True)

        for cp in lin_copies(0, 0):
            cp.start()
        for cp in lin_copies(1, 1):
            cp.start()
        for cp in lin_copies(0, 0):
            cp.wait()
        for cp in gat_copies(0):
            cp.start()

        def outer(i, carry):
            for b in (0, 1):
                j = 2 * i + b
                for cp in lin_copies(j + 1, 1 - b):
                    cp.wait()
                for cp in gat_copies(1 - b):
                    cp.start()
                for cp in gat_copies(b):
                    cp.wait()
                compute_scatter(b, _C)

                @pl.when(j <= nfull - 3)
                def _():
                    for cp in lin_copies(j + 2, b):
                        cp.start()
            return carry

        lax.fori_loop(0, (nfull - 1) // 2, outer, 0)
        lastb = (nfull - 1) % 2
        for cp in gat_copies(lastb):
            cp.wait()
        compute_scatter(lastb, _C)
        if tail:
            tb = base_w + nfull * _C
            pltpu.sync_copy(src_hbm.at[pl.ds(tb, tail)], src_t)
            pltpu.sync_copy(dst_hbm.at[pl.ds(tb, tail)],
                            dst0.at[pl.ds(0, tail)])
            pltpu.sync_copy(feat_hbm.at[pl.ds(tb, tail)],
                            feat0.at[pl.ds(0, tail)])
            pltpu.async_copy(a_hbm.at[src_t], a0.at[pl.ds(0, tail)],
                             sg0).wait()
            pltpu.async_copy(d_hbm.at[dst0.at[pl.ds(0, tail)]],
                             d0.at[pl.ds(0, tail)], sg0).wait()
            compute_scatter(0, tail, idx_ref=src_t)

        plsc.subcore_barrier()
        pltpu.sync_copy(acc.at[pl.ds(s * rows_per, rows_per)],
                        part_hbm.at[c, pl.ds(s * rows_per, rows_per)])

    return k(a_tab, d_tab, feat, src, dst, zeros)


def _post_body(part_ref, wo0_ref, wo1_ref, out0_ref, out1_ref):
    acc = part_ref[0] + part_ref[1]
    out0_ref[...] = jnp.dot(acc[:, 0:16], wo0_ref[...],
                            preferred_element_type=jnp.float32)
    wo1 = wo1_ref[...]
    for k in range(3):
        out1_ref[k] = jnp.dot(acc[:, 16 * (k + 1):16 * (k + 2)], wo1,
                              preferred_element_type=jnp.float32)


def _post(part, w_o0, w_o1, n):
    npad = part.shape[1]
    nb = 2048
    grid = (npad // nb,)
    full = lambda shape: pl.BlockSpec(shape, lambda i: tuple(0 for _ in shape))
    return pl.pallas_call(
        _post_body,
        grid=grid,
        in_specs=[
            pl.BlockSpec((2, nb, 64), lambda i: (0, i, 0)),
            full((RANK, 128)),
            full((RANK, 128)),
        ],
        out_specs=[
            pl.BlockSpec((nb, 128), lambda i: (i, 0)),
            pl.BlockSpec((3, nb, 128), lambda i: (0, i, 0)),
        ],
        out_shape=[
            jax.ShapeDtypeStruct((n, 128), jnp.float32),
            jax.ShapeDtypeStruct((3, n, 128), jnp.float32),
        ],
    )(part, w_o0, w_o1)


def kernel(graph, r_ij, z_0, z_1, emb, W_enc, b_enc, W_src, b_src, W_dst,
           b_dst, W_z0, W_z1, W_g, W_o0, W_o1):
    src = graph[0]
    dst = graph[1]
    n = z_0.shape[0]
    a_tab, d_tab = _prep_nodes(emb, z_0, z_1[:, 0, :], z_1[:, 1, :],
                               z_1[:, 2, :], W_src, W_dst, W_g, W_z0, W_z1)
    feat = _prep_edges(r_ij.T, W_enc, W_g, b_enc, b_src, b_dst)
    npad = -(-n // 2048) * 2048
    zeros = jnp.zeros((npad // 16, 64), jnp.float32)
    part = _sc_edges(a_tab, d_tab, feat, src, dst, zeros)
    out0, out1t = _post(part, W_o0, W_o1, n)
    return out0, out1t.transpose(1, 0, 2)
